# baseline (speedup 1.0000x reference)
import jax
import jax.numpy as jnp
from jax.experimental import pallas as pl
from jax.experimental.pallas import tpu as pltpu

_B, _T, _P, _H = 8, 4, 1025, 1280
_ROWS = 9


def _body(ids_ref, gate_ref, hid_ref, emb_ref, tile_ref, out_ref):
    tg = jnp.tanh(gate_ref[0])
    out_ref[0, 0] = (hid_ref[0, 0] + (1.0 - tg) * emb_ref[...]) + tg * tile_ref[0, 0]


def kernel(hidden_state, aspect_ratio_ids, gate, embedding, tile_table):
    ids = aspect_ratio_ids.astype(jnp.int32)
    tile4 = tile_table.reshape(_ROWS, _T, _P, _H)
    kfn = pl.pallas_call(
        _body,
        grid_spec=pltpu.PrefetchScalarGridSpec(
            num_scalar_prefetch=1,
            grid=(_B, _T),
            in_specs=[
                pl.BlockSpec(memory_space=pltpu.SMEM),
                pl.BlockSpec((1, 1, _P, _H), lambda b, t, ids: (b, t, 0, 0)),
                pl.BlockSpec((_P, _H), lambda b, t, ids: (0, 0)),
                pl.BlockSpec((1, 1, _P, _H), lambda b, t, ids: (ids[b], t, 0, 0)),
            ],
            out_specs=pl.BlockSpec((1, 1, _P, _H), lambda b, t, ids: (b, t, 0, 0)),
        ),
        out_shape=jax.ShapeDtypeStruct(hidden_state.shape, hidden_state.dtype),
        compiler_params=pltpu.CompilerParams(
            dimension_semantics=("arbitrary", "arbitrary"),
            vmem_limit_bytes=100 * 1024 * 1024,
        ),
    )
    return kfn(ids, gate, hidden_state, embedding, tile4)
